# baseline (speedup 1.0000x reference)
import functools

import jax
import jax.numpy as jnp
from jax import lax
from jax.experimental import pallas as pl
from jax.experimental.pallas import tpu as pltpu
from jax.experimental.pallas import tpu_sc as plsc

_NC = 2
_NS = 16
_NW = _NC * _NS
_K = 16
_INF = 3.0e38

_LEVELS = [
    (1024, 1024, 32, 64),
    (1024, 512, 64, 128),
    (512, 256, 128, 256),
    (256, 128, 256, 512),
]


def _pad128(c):
    return max(128, c)


def _prep_body(N, x_ref, xT_ref, wle_ref, feat_ref, i1_ref):
    b = pl.program_id(0)
    xyz = x_ref[0]
    xyzT = xT_ref[0]

    feat_ref[0] = jax.nn.gelu(
        jnp.dot(xyz, wle_ref[...], preferred_element_type=jnp.float32))

    n_pt = jnp.sum(xyz * xyz, axis=1, keepdims=True)
    n_ct = jnp.sum(xyzT * xyzT, axis=0, keepdims=True)

    for (n, s, _, _), out_ref in zip((_LEVELS[0],), (i1_ref,)):
        cross = jnp.dot(xyz[:n, :], xyzT[:, :s],
                        preferred_element_type=jnp.float32)
        d2 = n_ct[:, :s] + n_pt[:n, :] - 2.0 * cross
        iota_n = lax.broadcasted_iota(jnp.int32, (n, s), 0)
        rows = []
        for _ in range(_K):
            m = jnp.min(d2, axis=0, keepdims=True)
            am = jnp.min(jnp.where(d2 <= m, iota_n, n), axis=0,
                         keepdims=True)
            rows.append(am)
            d2 = jnp.where(iota_n == am, _INF, d2)
        out_ref[0] = jnp.concatenate(rows, axis=0) + b * n


def _prep34_body(x_ref, xT_ref, i3_ref, i4_ref):
    b = pl.program_id(0)
    xyz = x_ref[0]
    xyzT = xT_ref[0]
    n_pt = jnp.sum(xyz * xyz, axis=1, keepdims=True)
    n_ct = jnp.sum(xyzT * xyzT, axis=0, keepdims=True)
    for (n, s, _, _), out_ref in zip((_LEVELS[2], _LEVELS[3]),
                                     (i3_ref, i4_ref)):
        cross = jnp.dot(xyz[:n, :], xyzT[:, :s],
                        preferred_element_type=jnp.float32)
        d2 = n_ct[:, :s] + n_pt[:n, :] - 2.0 * cross
        iota_n = lax.broadcasted_iota(jnp.int32, (n, s), 0)
        rows = []
        for _ in range(_K):
            m = jnp.min(d2, axis=0, keepdims=True)
            am = jnp.min(jnp.where(d2 <= m, iota_n, n), axis=0,
                         keepdims=True)
            rows.append(am)
            d2 = jnp.where(iota_n == am, _INF, d2)
        out_ref[0] = jnp.concatenate(rows, axis=0) + b * n


def _make_prep34(B):
    return pl.pallas_call(
        _prep34_body,
        grid=(B,),
        in_specs=[
            pl.BlockSpec((1, 512, 3), lambda b: (b, 0, 0)),
            pl.BlockSpec((1, 3, 512), lambda b: (b, 0, 0)),
        ],
        out_specs=[
            pl.BlockSpec((1, _K, s), lambda b, _s=s: (b, 0, 0))
            for (_, s, _, _) in (_LEVELS[2], _LEVELS[3])
        ],
        out_shape=[
            jax.ShapeDtypeStruct((B, _K, s), jnp.int32)
            for (_, s, _, _) in (_LEVELS[2], _LEVELS[3])
        ],
    )


def _make_prep(B, N):
    return pl.pallas_call(
        functools.partial(_prep_body, N),
        grid=(B,),
        in_specs=[
            pl.BlockSpec((1, N, 3), lambda b: (b, 0, 0)),
            pl.BlockSpec((1, 3, N), lambda b: (b, 0, 0)),
            pl.BlockSpec((3, 128), lambda b: (0, 0)),
        ],
        out_specs=[
            pl.BlockSpec((1, N, 128), lambda b: (b, 0, 0)),
            pl.BlockSpec((1, _K, N), lambda b: (b, 0, 0)),
        ],
        out_shape=[
            jax.ShapeDtypeStruct((B, N, 128), jnp.float32),
            jax.ShapeDtypeStruct((B, _K, N), jnp.int32),
        ],
    )


def _make_sc_gather(B, R, NR, C):
    rpw = NR // _NW
    rc = 128
    while rpw % rc:
        rc //= 2
    nchunks = rpw // rc
    assert nchunks % 2 == 0 and _NW % B == 0 and C % 128 == 0
    wpb = _NW // B
    rpb = NR // B
    mesh = plsc.VectorSubcoreMesh(core_axis_name="c", subcore_axis_name="s")

    @functools.partial(
        pl.kernel,
        out_type=jax.ShapeDtypeStruct((B, rpb, C), jnp.float32),
        mesh=mesh,
        scratch_types=[
            pltpu.VMEM((rpw,), jnp.int32),
            pltpu.VMEM((rc, C), jnp.float32),
            pltpu.VMEM((rc, C), jnp.float32),
            pltpu.SemaphoreType.DMA,
            pltpu.SemaphoreType.DMA,
        ],
    )
    def sck(table_hbm, idx_hbm, out_hbm, idx_v, rows0_v, rows1_v,
            sem0, sem1):
        wid = lax.axis_index("s") * _NC + lax.axis_index("c")
        base_r = wid * rpw
        bat = wid // wpb
        base_in_b = (wid % wpb) * rpw
        bufs = (rows0_v, rows1_v)
        sems = (sem0, sem1)

        pltpu.sync_copy(idx_hbm.at[pl.ds(base_r, rpw)], idx_v)
        pltpu.async_copy(
            table_hbm.at[idx_v.at[pl.ds(0, rc)]], rows0_v, sem0)

        def pair(i, carry):
            for j in range(2):
                cur = i * 2 + j
                nxt = cur + 1

                @pl.when(nxt < nchunks)
                def _():
                    pltpu.async_copy(
                        table_hbm.at[idx_v.at[pl.ds(nxt * rc, rc)]],
                        bufs[1 - j], sems[1 - j])

                pltpu.make_async_copy(
                    table_hbm.at[idx_v.at[pl.ds(cur * rc, rc)]],
                    bufs[j], sems[j]).wait()
                pltpu.sync_copy(
                    bufs[j],
                    out_hbm.at[bat, pl.ds(base_in_b + cur * rc, rc)])
            return carry

        lax.fori_loop(0, nchunks // 2, pair, 0)

    return sck


def _conv_body(S, C, CT, gath_ref, cent_ref, w_ref, out_ref):
    gath = jnp.reshape(gath_ref[0], (_K, S, CT))[:, :, :C]
    cent = cent_ref[0][:S, :C]
    edge = jnp.concatenate([gath, gath - cent[None, :, :]], axis=-1)
    edge2 = jnp.reshape(edge, (_K * S, 2 * C))
    t = jnp.dot(edge2, w_ref[...], preferred_element_type=jnp.float32)
    t3 = jnp.reshape(t, (_K, S, t.shape[-1]))
    out_ref[0] = jax.nn.gelu(jnp.max(t3, axis=0))


def _make_conv(B, N, S, C, CT, D, DT):
    return pl.pallas_call(
        functools.partial(_conv_body, S, C, CT),
        grid=(B,),
        in_specs=[
            pl.BlockSpec((1, _K * S, CT), lambda b: (b, 0, 0)),
            pl.BlockSpec((1, N, CT), lambda b: (b, 0, 0)),
            pl.BlockSpec((2 * C, DT), lambda b: (0, 0)),
        ],
        out_specs=pl.BlockSpec((1, S, DT), lambda b: (b, 0, 0)),
        out_shape=jax.ShapeDtypeStruct((B, S, DT), jnp.float32),
    )


def _conv_tail_body(S, C, gath_ref, cent_ref, w_ref, wlast_ref, wlin1_ref,
                    bng_ref, bnb_ref, wlin2_ref, blin2_ref, out_ref):
    gath = jnp.reshape(gath_ref[0], (_K, S, C))
    cent = cent_ref[0][:S, :]
    edge = jnp.concatenate([gath, gath - cent[None, :, :]], axis=-1)
    edge2 = jnp.reshape(edge, (_K * S, 2 * C))
    t = jnp.dot(edge2, w_ref[...], preferred_element_type=jnp.float32)
    t3 = jnp.reshape(t, (_K, S, t.shape[-1]))
    f4 = jax.nn.gelu(jnp.max(t3, axis=0))

    h = jax.nn.gelu(jnp.dot(f4, wlast_ref[...],
                            preferred_element_type=jnp.float32))
    hmax = jnp.max(h, axis=0, keepdims=True)
    havg = jnp.mean(h, axis=0, keepdims=True)
    g = jnp.concatenate([hmax, havg], axis=-1)
    g = jnp.dot(g, wlin1_ref[...], preferred_element_type=jnp.float32)
    g = g * bng_ref[...] + bnb_ref[...]
    g = jax.nn.gelu(g)
    out_ref[0] = (jnp.dot(g, wlin2_ref[...],
                          preferred_element_type=jnp.float32)
                  + blin2_ref[...])


def _make_conv_tail(B, N, S, C):
    return pl.pallas_call(
        functools.partial(_conv_tail_body, S, C),
        grid=(B,),
        in_specs=[
            pl.BlockSpec((1, _K * S, C), lambda b: (b, 0, 0)),
            pl.BlockSpec((1, N, C), lambda b: (b, 0, 0)),
            pl.BlockSpec((2 * C, 512), lambda b: (0, 0)),
            pl.BlockSpec((512, 1024), lambda b: (0, 0)),
            pl.BlockSpec((2048, 512), lambda b: (0, 0)),
            pl.BlockSpec((1, 512), lambda b: (0, 0)),
            pl.BlockSpec((1, 512), lambda b: (0, 0)),
            pl.BlockSpec((512, 40), lambda b: (0, 0)),
            pl.BlockSpec((1, 40), lambda b: (0, 0)),
        ],
        out_specs=pl.BlockSpec((1, 1, 40), lambda b: (b, 0, 0)),
        out_shape=jax.ShapeDtypeStruct((B, 1, 40), jnp.float32),
    )


def kernel(x, W_le, W1, W2, W3, W4, W_last, W_lin1, bn_g, bn_b,
           W_lin2, b_lin2):
    B, N, _ = x.shape
    f32 = jnp.float32

    xT = jnp.transpose(x, (0, 2, 1))
    wle_p = jnp.zeros((3, 128), f32).at[:, :32].set(W_le)
    w1_p = jnp.zeros((64, 128), f32).at[:, :64].set(W1)

    feat, i1 = _make_prep(B, N)(x, xT, wle_p)
    i2 = i1[:, :, :512]
    ws = (w1_p, W2, W3, W4)

    idxs = [i1, i2, None, None]
    for li, (n, s, c, d) in enumerate(_LEVELS[:3]):
        ct = _pad128(c)
        dt = _pad128(d)
        gath = _make_sc_gather(B, B * n, B * s * _K, ct)(
            jnp.reshape(feat, (B * n, ct)),
            jnp.reshape(idxs[li], (B * s * _K,)))
        if li == 0:
            i3, i4 = _make_prep34(B)(x[:, :512, :], xT[:, :, :512])
            idxs[2], idxs[3] = i3, i4
        feat = _make_conv(B, n, s, c, ct, d, dt)(gath, feat, ws[li])

    n, s, c, d = _LEVELS[3]
    gath = _make_sc_gather(B, B * n, B * s * _K, c)(
        jnp.reshape(feat, (B * n, c)),
        jnp.reshape(idxs[3], (B * s * _K,)))
    out = _make_conv_tail(B, n, s, c)(
        gath, feat, W4, W_last, W_lin1,
        jnp.reshape(bn_g, (1, -1)), jnp.reshape(bn_b, (1, -1)),
        W_lin2, jnp.reshape(b_lin2, (1, -1)))
    return jnp.reshape(out, (B, 40))
